# baseline (speedup 1.0000x reference)
import jax
import jax.numpy as jnp
from jax.experimental import pallas as pl
from jax.experimental.pallas import tpu as pltpu

N = 10000
NFEAT = 128
NHID = 64
BI = 400
BI2 = 400

QDT = jnp.float8_e5m2


def _pass1_kernel(adj_ref, x_ref, w1_ref, b1_ref, w2_ref,
                  q_ref, x1_ref, s2_ref, s1_scr):
    i = pl.program_id(0)

    @pl.when(i == 0)
    def _():
        s1 = jnp.dot(x_ref[...], w1_ref[...], preferred_element_type=jnp.float32)
        s1_scr[...] = s1.astype(jnp.bfloat16)

    a16 = adj_ref[...].astype(jnp.bfloat16)
    q_ref[...] = a16.astype(QDT)
    acc = jnp.dot(a16, s1_scr[...], preferred_element_type=jnp.float32)
    x1 = jnp.maximum(acc + b1_ref[...], 0.0)
    x1_ref[...] = x1
    s2_ref[...] = jnp.dot(x1, w2_ref[...],
                          preferred_element_type=jnp.float32).astype(QDT)


def _pass2_kernel(q_ref, s2_ref, b2_ref, w3_ref, x2_ref, s3_ref):
    acc = jnp.dot(q_ref[...], s2_ref[...], preferred_element_type=jnp.float32)
    x2 = jnp.maximum(acc + b2_ref[...], 0.0)
    x2_ref[...] = x2
    s3_ref[...] = jnp.dot(x2, w3_ref[...],
                          preferred_element_type=jnp.float32).astype(QDT)


def _pass3_kernel(q_ref, s3_ref, x1_ref, x2_ref, b3_ref,
                  wl1_ref, wl2_ref, wl3_ref, bl_ref, out_ref):
    x3 = (jnp.dot(q_ref[...], s3_ref[...],
                  preferred_element_type=jnp.float32) + b3_ref[...])
    logits = (jnp.dot(x1_ref[...], wl1_ref[...], preferred_element_type=jnp.float32)
              + jnp.dot(x2_ref[...], wl2_ref[...], preferred_element_type=jnp.float32)
              + jnp.dot(x3, wl3_ref[...], preferred_element_type=jnp.float32)
              + bl_ref[...])
    m = jnp.max(logits, axis=1, keepdims=True)
    lse = jnp.log(jnp.sum(jnp.exp(logits - m), axis=1, keepdims=True)) + m
    out_ref[...] = logits - lse


def kernel(x, adj, W1, b1, W2, b2, W3, b3, Wl, bl):
    nclasses = Wl.shape[1]
    b1r = b1.reshape(1, NHID)
    b2r = b2.reshape(1, NHID)
    b3r = b3.reshape(1, NHID)
    blr = bl.reshape(1, nclasses)
    Wl1, Wl2, Wl3 = Wl[:NHID], Wl[NHID:2 * NHID], Wl[2 * NHID:]

    const = lambda *_: (0, 0)
    params = pltpu.CompilerParams(dimension_semantics=("arbitrary",))

    q, x1, s2 = pl.pallas_call(
        _pass1_kernel,
        grid=(N // BI,),
        in_specs=[
            pl.BlockSpec((BI, N), lambda i: (i, 0)),
            pl.BlockSpec((N, NFEAT), const),
            pl.BlockSpec((NFEAT, NHID), const),
            pl.BlockSpec((1, NHID), const),
            pl.BlockSpec((NHID, NHID), const),
        ],
        out_specs=[
            pl.BlockSpec((BI, N), lambda i: (i, 0)),
            pl.BlockSpec((BI, NHID), lambda i: (i, 0)),
            pl.BlockSpec((BI, NHID), lambda i: (i, 0)),
        ],
        out_shape=[
            jax.ShapeDtypeStruct((N, N), QDT),
            jax.ShapeDtypeStruct((N, NHID), jnp.float32),
            jax.ShapeDtypeStruct((N, NHID), QDT),
        ],
        scratch_shapes=[pltpu.VMEM((N, NHID), jnp.bfloat16)],
        compiler_params=params,
    )(adj, x, W1, b1r, W2)

    x2, s3 = pl.pallas_call(
        _pass2_kernel,
        grid=(N // BI2,),
        in_specs=[
            pl.BlockSpec((BI2, N), lambda i: (i, 0)),
            pl.BlockSpec((N, NHID), const),
            pl.BlockSpec((1, NHID), const),
            pl.BlockSpec((NHID, NHID), const),
        ],
        out_specs=[
            pl.BlockSpec((BI2, NHID), lambda i: (i, 0)),
            pl.BlockSpec((BI2, NHID), lambda i: (i, 0)),
        ],
        out_shape=[
            jax.ShapeDtypeStruct((N, NHID), jnp.float32),
            jax.ShapeDtypeStruct((N, NHID), QDT),
        ],
        compiler_params=params,
    )(q, s2, b2r, W3)

    out = pl.pallas_call(
        _pass3_kernel,
        grid=(N // BI2,),
        in_specs=[
            pl.BlockSpec((BI2, N), lambda i: (i, 0)),
            pl.BlockSpec((N, NHID), const),
            pl.BlockSpec((BI2, NHID), lambda i: (i, 0)),
            pl.BlockSpec((BI2, NHID), lambda i: (i, 0)),
            pl.BlockSpec((1, NHID), const),
            pl.BlockSpec((NHID, nclasses), const),
            pl.BlockSpec((NHID, nclasses), const),
            pl.BlockSpec((NHID, nclasses), const),
            pl.BlockSpec((1, nclasses), const),
        ],
        out_specs=pl.BlockSpec((BI2, nclasses), lambda i: (i, 0)),
        out_shape=jax.ShapeDtypeStruct((N, nclasses), jnp.float32),
        compiler_params=params,
    )(q, s3, x1, x2, b3r, Wl1, Wl2, Wl3, blr)

    return out


# speedup vs baseline: 1.0884x; 1.0884x over previous
import jax
import jax.numpy as jnp
from jax.experimental import pallas as pl
from jax.experimental.pallas import tpu as pltpu

N = 10000
NFEAT = 128
NHID = 64
BI = 400
BI2 = 1000

QDT = jnp.float8_e5m2


def _pass1_kernel(adj_ref, x_ref, w1_ref, b1_ref, w2_ref,
                  q_ref, x1_ref, s2_ref, s1_scr):
    i = pl.program_id(0)

    @pl.when(i == 0)
    def _():
        s1 = jnp.dot(x_ref[...], w1_ref[...], preferred_element_type=jnp.float32)
        s1_scr[...] = s1.astype(jnp.bfloat16)

    a16 = adj_ref[...].astype(jnp.bfloat16)
    q_ref[...] = a16.astype(QDT)
    acc = jnp.dot(a16, s1_scr[...], preferred_element_type=jnp.float32)
    x1 = jnp.maximum(acc + b1_ref[...], 0.0)
    x1_ref[...] = x1
    s2_ref[...] = jnp.dot(x1, w2_ref[...],
                          preferred_element_type=jnp.float32).astype(QDT)


def _pass2_kernel(q_ref, s2_ref, b2_ref, w3_ref, x2_ref, s3_ref):
    acc = jnp.dot(q_ref[...], s2_ref[...], preferred_element_type=jnp.float32)
    x2 = jnp.maximum(acc + b2_ref[...], 0.0)
    x2_ref[...] = x2
    s3_ref[...] = jnp.dot(x2, w3_ref[...],
                          preferred_element_type=jnp.float32).astype(QDT)


def _pass3_kernel(q_ref, s3_ref, x1_ref, x2_ref, b3_ref,
                  wl1_ref, wl2_ref, wl3_ref, bl_ref, out_ref):
    x3 = (jnp.dot(q_ref[...], s3_ref[...],
                  preferred_element_type=jnp.float32) + b3_ref[...])
    logits = (jnp.dot(x1_ref[...], wl1_ref[...], preferred_element_type=jnp.float32)
              + jnp.dot(x2_ref[...], wl2_ref[...], preferred_element_type=jnp.float32)
              + jnp.dot(x3, wl3_ref[...], preferred_element_type=jnp.float32)
              + bl_ref[...])
    m = jnp.max(logits, axis=1, keepdims=True)
    lse = jnp.log(jnp.sum(jnp.exp(logits - m), axis=1, keepdims=True)) + m
    out_ref[...] = logits - lse


def kernel(x, adj, W1, b1, W2, b2, W3, b3, Wl, bl):
    nclasses = Wl.shape[1]
    b1r = b1.reshape(1, NHID)
    b2r = b2.reshape(1, NHID)
    b3r = b3.reshape(1, NHID)
    blr = bl.reshape(1, nclasses)
    Wl1, Wl2, Wl3 = Wl[:NHID], Wl[NHID:2 * NHID], Wl[2 * NHID:]

    const = lambda *_: (0, 0)
    params = pltpu.CompilerParams(dimension_semantics=("arbitrary",))

    q, x1, s2 = pl.pallas_call(
        _pass1_kernel,
        grid=(N // BI,),
        in_specs=[
            pl.BlockSpec((BI, N), lambda i: (i, 0)),
            pl.BlockSpec((N, NFEAT), const),
            pl.BlockSpec((NFEAT, NHID), const),
            pl.BlockSpec((1, NHID), const),
            pl.BlockSpec((NHID, NHID), const),
        ],
        out_specs=[
            pl.BlockSpec((BI, N), lambda i: (i, 0)),
            pl.BlockSpec((BI, NHID), lambda i: (i, 0)),
            pl.BlockSpec((BI, NHID), lambda i: (i, 0)),
        ],
        out_shape=[
            jax.ShapeDtypeStruct((N, N), QDT),
            jax.ShapeDtypeStruct((N, NHID), jnp.float32),
            jax.ShapeDtypeStruct((N, NHID), QDT),
        ],
        scratch_shapes=[pltpu.VMEM((N, NHID), jnp.bfloat16)],
        compiler_params=params,
    )(adj, x, W1, b1r, W2)

    x2, s3 = pl.pallas_call(
        _pass2_kernel,
        grid=(N // BI2,),
        in_specs=[
            pl.BlockSpec((BI2, N), lambda i: (i, 0)),
            pl.BlockSpec((N, NHID), const),
            pl.BlockSpec((1, NHID), const),
            pl.BlockSpec((NHID, NHID), const),
        ],
        out_specs=[
            pl.BlockSpec((BI2, NHID), lambda i: (i, 0)),
            pl.BlockSpec((BI2, NHID), lambda i: (i, 0)),
        ],
        out_shape=[
            jax.ShapeDtypeStruct((N, NHID), jnp.float32),
            jax.ShapeDtypeStruct((N, NHID), QDT),
        ],
        compiler_params=params,
    )(q, s2, b2r, W3)

    out = pl.pallas_call(
        _pass3_kernel,
        grid=(N // BI2,),
        in_specs=[
            pl.BlockSpec((BI2, N), lambda i: (i, 0)),
            pl.BlockSpec((N, NHID), const),
            pl.BlockSpec((BI2, NHID), lambda i: (i, 0)),
            pl.BlockSpec((BI2, NHID), lambda i: (i, 0)),
            pl.BlockSpec((1, NHID), const),
            pl.BlockSpec((NHID, nclasses), const),
            pl.BlockSpec((NHID, nclasses), const),
            pl.BlockSpec((NHID, nclasses), const),
            pl.BlockSpec((1, nclasses), const),
        ],
        out_specs=pl.BlockSpec((BI2, nclasses), lambda i: (i, 0)),
        out_shape=jax.ShapeDtypeStruct((N, nclasses), jnp.float32),
        compiler_params=params,
    )(q, s3, x1, x2, b3r, Wl1, Wl2, Wl3, blr)

    return out


# speedup vs baseline: 1.1070x; 1.0171x over previous
import jax
import jax.numpy as jnp
from jax.experimental import pallas as pl
from jax.experimental.pallas import tpu as pltpu

N = 10000
NFEAT = 128
NHID = 64
BI = 400
BI2 = 1000

QDT = jnp.float8_e5m2


def _pass1_kernel(adj_ref, x_ref, w1_ref, b1_ref, w2_ref,
                  q_ref, x1_ref, s2_ref, s1_scr):
    i = pl.program_id(0)

    @pl.when(i == 0)
    def _():
        s1 = jnp.dot(x_ref[...], w1_ref[...], preferred_element_type=jnp.float32)
        s1_scr[...] = s1.astype(jnp.bfloat16)

    a16 = adj_ref[...].astype(jnp.bfloat16)
    q_ref[...] = a16.astype(QDT)
    acc = jnp.dot(a16, s1_scr[...], preferred_element_type=jnp.float32)
    x1 = jnp.maximum(acc + b1_ref[...], 0.0)
    x1_ref[...] = x1
    s2_ref[...] = jnp.dot(x1, w2_ref[...],
                          preferred_element_type=jnp.float32).astype(QDT)


def _pass23_kernel(q_ref, s2_ref, x1_ref, b2_ref, w3_ref, b3_ref,
                   wl1_ref, wl2_ref, wl3_ref, bl_ref, out_ref,
                   x2_scr, s3_scr):
    p = pl.program_id(0)
    i = pl.program_id(1)
    rows = pl.ds(i * BI2, BI2)

    @pl.when(p == 0)
    def _():
        acc = jnp.dot(q_ref[...], s2_ref[...], preferred_element_type=jnp.float32)
        x2 = jnp.maximum(acc + b2_ref[...], 0.0)
        x2_scr[rows, :] = x2
        s3_scr[rows, :] = jnp.dot(x2, w3_ref[...],
                                  preferred_element_type=jnp.float32).astype(QDT)

    @pl.when(p == 1)
    def _():
        x3 = (jnp.dot(q_ref[...], s3_scr[...],
                      preferred_element_type=jnp.float32) + b3_ref[...])
        logits = (jnp.dot(x1_ref[...], wl1_ref[...], preferred_element_type=jnp.float32)
                  + jnp.dot(x2_scr[rows, :], wl2_ref[...], preferred_element_type=jnp.float32)
                  + jnp.dot(x3, wl3_ref[...], preferred_element_type=jnp.float32)
                  + bl_ref[...])
        m = jnp.max(logits, axis=1, keepdims=True)
        lse = jnp.log(jnp.sum(jnp.exp(logits - m), axis=1, keepdims=True)) + m
        out_ref[0] = logits - lse


def kernel(x, adj, W1, b1, W2, b2, W3, b3, Wl, bl):
    nclasses = Wl.shape[1]
    b1r = b1.reshape(1, NHID)
    b2r = b2.reshape(1, NHID)
    b3r = b3.reshape(1, NHID)
    blr = bl.reshape(1, nclasses)
    Wl1, Wl2, Wl3 = Wl[:NHID], Wl[NHID:2 * NHID], Wl[2 * NHID:]

    const = lambda *_: (0, 0)
    params = pltpu.CompilerParams(dimension_semantics=("arbitrary",))

    q, x1, s2 = pl.pallas_call(
        _pass1_kernel,
        grid=(N // BI,),
        in_specs=[
            pl.BlockSpec((BI, N), lambda i: (i, 0)),
            pl.BlockSpec((N, NFEAT), const),
            pl.BlockSpec((NFEAT, NHID), const),
            pl.BlockSpec((1, NHID), const),
            pl.BlockSpec((NHID, NHID), const),
        ],
        out_specs=[
            pl.BlockSpec((BI, N), lambda i: (i, 0)),
            pl.BlockSpec((BI, NHID), lambda i: (i, 0)),
            pl.BlockSpec((BI, NHID), lambda i: (i, 0)),
        ],
        out_shape=[
            jax.ShapeDtypeStruct((N, N), QDT),
            jax.ShapeDtypeStruct((N, NHID), jnp.float32),
            jax.ShapeDtypeStruct((N, NHID), QDT),
        ],
        scratch_shapes=[pltpu.VMEM((N, NHID), jnp.bfloat16)],
        compiler_params=params,
    )(adj, x, W1, b1r, W2)

    params2 = pltpu.CompilerParams(dimension_semantics=("arbitrary", "arbitrary"))
    out = pl.pallas_call(
        _pass23_kernel,
        grid=(2, N // BI2),
        in_specs=[
            pl.BlockSpec((BI2, N), lambda p, i: (i, 0)),
            pl.BlockSpec((N, NHID), const),
            pl.BlockSpec((BI2, NHID), lambda p, i: (i, 0)),
            pl.BlockSpec((1, NHID), const),
            pl.BlockSpec((NHID, NHID), const),
            pl.BlockSpec((1, NHID), const),
            pl.BlockSpec((NHID, nclasses), const),
            pl.BlockSpec((NHID, nclasses), const),
            pl.BlockSpec((NHID, nclasses), const),
            pl.BlockSpec((1, nclasses), const),
        ],
        out_specs=pl.BlockSpec((1, BI2, nclasses), lambda p, i: (p, i, 0)),
        out_shape=jax.ShapeDtypeStruct((2, N, nclasses), jnp.float32),
        scratch_shapes=[
            pltpu.VMEM((N, NHID), jnp.float32),
            pltpu.VMEM((N, NHID), QDT),
        ],
        compiler_params=params2,
    )(q, s2, x1, b2r, W3, b3r, Wl1, Wl2, Wl3, blr)

    return out[1]
